# baseline (speedup 1.0000x reference)
import jax
from jax.experimental import pallas as pl
from jax.experimental.pallas import tpu as pltpu

_BLOCK_ROWS = 5000


def _copy_kernel(x_hbm, rel_hbm, x_out_blk, rel_out_blk, sem, rel_sem):
    i = pl.program_id(0)
    load = pltpu.make_async_copy(
        x_hbm.at[pl.ds(i * _BLOCK_ROWS, _BLOCK_ROWS), :], x_out_blk, sem
    )
    load.start()

    @pl.when(i == 0)
    def _():
        rel_load = pltpu.make_async_copy(rel_hbm, rel_out_blk, rel_sem)
        rel_load.start()
        rel_load.wait()

    load.wait()


def kernel(x, edge_index, edge_type, rel_embed):
    n, d = x.shape
    r, _ = rel_embed.shape
    x_out, rel_out = pl.pallas_call(
        _copy_kernel,
        grid=(n // _BLOCK_ROWS,),
        in_specs=[
            pl.BlockSpec(memory_space=pl.MemorySpace.ANY),
            pl.BlockSpec(memory_space=pl.MemorySpace.ANY),
        ],
        out_specs=[
            pl.BlockSpec((_BLOCK_ROWS, d), lambda i: (i, 0)),
            pl.BlockSpec((r, d), lambda i: (0, 0)),
        ],
        out_shape=[
            jax.ShapeDtypeStruct(x.shape, x.dtype),
            jax.ShapeDtypeStruct(rel_embed.shape, rel_embed.dtype),
        ],
        scratch_shapes=[
            pltpu.SemaphoreType.DMA,
            pltpu.SemaphoreType.DMA,
        ],
        compiler_params=pltpu.CompilerParams(
            dimension_semantics=("arbitrary",),
        ),
    )(x, rel_embed)
    return (x_out, rel_out)
